# baseline (speedup 1.0000x reference)
import functools

import jax
import jax.numpy as jnp
import numpy as np
from jax import lax
from jax.experimental import pallas as pl
from jax.experimental.pallas import tpu as pltpu
from jax.experimental.pallas import tpu_sc as plsc
from jax.experimental.shard_map import shard_map
from jax.sharding import Mesh, PartitionSpec as P

D = 256
KC = 8192
N = 16384
CCOST = 0.25
EPS = 1e-12

BN = 128
NTILES = 32
CHUNK = 64


def _dist_body(xn_ref, wnT_ref, b_ref, a_ref, idx_ref, wb_ref):
    xn = xn_ref[...]
    s = jnp.dot(xn, wnT_ref[...], preferred_element_type=jnp.float32)
    d = (a_ref[...] + b_ref[...]) - 2.0 * s
    it = lax.broadcasted_iota(jnp.int32, d.shape, 1)
    inf = jnp.float32(jnp.inf)
    m1 = jnp.min(d, axis=1, keepdims=True)
    i1 = jnp.min(jnp.where(d == m1, it, KC), axis=1, keepdims=True)
    d2 = jnp.where(it == i1, inf, d)
    m2 = jnp.min(d2, axis=1, keepdims=True)
    i2 = jnp.min(jnp.where(d2 == m2, it, KC), axis=1, keepdims=True)
    d3 = jnp.where(it == i2, inf, d2)
    m3 = jnp.min(d3, axis=1, keepdims=True)
    i3 = jnp.min(jnp.where(d3 == m3, it, KC), axis=1, keepdims=True)
    idx_ref[...] = jnp.concatenate([i1, i2, i3, i3], axis=1)
    inv1 = 1.0 / m1
    inv2 = 1.0 / m2
    inv3 = 1.0 / m3
    tot = (inv1 + inv2) + inv3
    one16 = jnp.ones((xn.shape[0], 16), jnp.float32)
    wb_ref[...] = jnp.concatenate(
        [(inv1 / tot) * one16, (inv2 / tot) * one16, (inv3 / tot) * one16],
        axis=1)


def _sc_combine(x, emb, idx_flat, wb, n_tok):
    info = plsc.get_sparse_core_info()
    mesh = plsc.VectorSubcoreMesh(core_axis_name="c", subcore_axis_name="s")
    tpt = n_tok // NTILES
    nchunk = tpt // CHUNK

    @functools.partial(
        pl.kernel,
        out_type=(jax.ShapeDtypeStruct((n_tok, D), jnp.float32),
                  jax.ShapeDtypeStruct((NTILES, 16), jnp.float32)),
        mesh=mesh,
        scratch_types=[
            pltpu.VMEM((96,), jnp.int32),
            pltpu.VMEM((96,), jnp.int32),
            pltpu.VMEM((3 * CHUNK, D), jnp.float32),
            pltpu.VMEM((CHUNK, D), jnp.float32),
            pltpu.VMEM((CHUNK, D), jnp.float32),
            pltpu.VMEM((CHUNK, 48), jnp.float32),
            pltpu.VMEM((16,), jnp.float32),
            pltpu.SemaphoreType.DMA,
            pltpu.SemaphoreType.DMA,
        ],
    )
    def k(x_hbm, emb_hbm, idxf_hbm, wb_hbm, q_hbm, lp_hbm,
          idxa, idxb, rows, xv, qv, wbv, accv, sem0, sem1):
        wid = lax.axis_index("s") * info.num_cores + lax.axis_index("c")
        accv[...] = jnp.zeros((16,), jnp.float32)

        @pl.loop(0, nchunk)
        def _chunk(c):
            tb = wid * tpt + c * CHUNK
            fb = 3 * tb
            pltpu.sync_copy(idxf_hbm.at[pl.ds(fb, 96)], idxa)
            pltpu.sync_copy(idxf_hbm.at[pl.ds(fb + 96, 96)], idxb)
            cp0 = pltpu.async_copy(emb_hbm.at[idxa], rows.at[pl.ds(0, 96)],
                                   sem0)
            cp1 = pltpu.async_copy(emb_hbm.at[idxb], rows.at[pl.ds(96, 96)],
                                   sem1)
            pltpu.sync_copy(x_hbm.at[pl.ds(tb, CHUNK)], xv)
            pltpu.sync_copy(wb_hbm.at[pl.ds(tb, CHUNK)], wbv)
            cp0.wait()
            cp1.wait()

            @pl.loop(0, CHUNK)
            def _tok(t):
                w0 = wbv[t, pl.ds(0, 16)]
                w1 = wbv[t, pl.ds(16, 16)]
                w2 = wbv[t, pl.ds(32, 16)]
                for v in range(D // 16):
                    sl = pl.ds(v * 16, 16)
                    r0 = rows[3 * t, sl]
                    r1 = rows[3 * t + 1, sl]
                    r2 = rows[3 * t + 2, sl]
                    q = (w0 * r0 + w1 * r1) + w2 * r2
                    xs = xv[t, sl]
                    dq = q - xs
                    qv[t, sl] = xs + dq
                    accv[...] = accv[...] + dq * dq

            pltpu.sync_copy(qv, q_hbm.at[pl.ds(tb, CHUNK)])

        pltpu.sync_copy(accv, lp_hbm.at[wid])

    return k(x, emb, idx_flat, wb)


def _pipeline(x, emb):
    n_tok = x.shape[0]
    xn = x / jnp.maximum(
        jnp.sqrt(jnp.sum(x * x, axis=1, keepdims=True)), EPS)
    wn = emb / jnp.maximum(
        jnp.sqrt(jnp.sum(emb * emb, axis=1, keepdims=True)), EPS)
    a = jnp.sum(xn ** 2, axis=1, keepdims=True)
    b = jnp.sum(wn ** 2, axis=1).reshape(1, KC)
    wnT = wn.T
    idx4, wb = pl.pallas_call(
        _dist_body,
        grid=(n_tok // BN,),
        in_specs=[pl.BlockSpec((BN, D), lambda i: (i, 0)),
                  pl.BlockSpec((D, KC), lambda i: (0, 0)),
                  pl.BlockSpec((1, KC), lambda i: (0, 0)),
                  pl.BlockSpec((BN, 1), lambda i: (i, 0))],
        out_specs=[pl.BlockSpec((BN, 4), lambda i: (i, 0)),
                   pl.BlockSpec((BN, 48), lambda i: (i, 0))],
        out_shape=(jax.ShapeDtypeStruct((n_tok, 4), jnp.int32),
                   jax.ShapeDtypeStruct((n_tok, 48), jnp.float32)),
    )(xn, wnT, b, a)
    top_idx = idx4[:, :3]
    idx_flat = top_idx.reshape(-1)
    q_st, lp = _sc_combine(x, emb, idx_flat, wb, n_tok)
    return q_st, jnp.sum(lp), top_idx


def kernel(x, emb):
    devs = jax.devices()[:2]
    mesh = Mesh(np.array(devs), ("dp",))

    def shard_fn(x_l, emb_l):
        q_st, lp_sum, top_idx = _pipeline(x_l, emb_l)
        m = lax.psum(lp_sum, "dp") / jnp.float32(N * D)
        loss = m + CCOST * m
        return q_st, loss, top_idx

    q_st, loss, top_idx = shard_map(
        shard_fn, mesh=mesh,
        in_specs=(P("dp", None), P(None, None)),
        out_specs=(P("dp", None), P(), P("dp", None)),
        check_rep=False,
    )(x, emb)
    return (q_st, loss, top_idx)


# speedup vs baseline: 1.2788x; 1.2788x over previous
import functools

import jax
import jax.numpy as jnp
from jax import lax
from jax.experimental import pallas as pl
from jax.experimental.pallas import tpu as pltpu
from jax.experimental.pallas import tpu_sc as plsc

D = 256
KC = 8192
KH = KC // 2
N = 16384
CCOST = 0.25
EPS = 1e-12

BN = 256
NTILES = 32
CHUNK = 64


def _top3(d, itf):
    inf = jnp.float32(jnp.inf)
    kcf = jnp.float32(d.shape[1])
    m1 = jnp.min(d, axis=1, keepdims=True)
    i1 = jnp.min(jnp.where(d == m1, itf, kcf), axis=1, keepdims=True)
    d2 = jnp.where(itf == i1, inf, d)
    m2 = jnp.min(d2, axis=1, keepdims=True)
    i2 = jnp.min(jnp.where(d2 == m2, itf, kcf), axis=1, keepdims=True)
    d3 = jnp.where(itf == i2, inf, d2)
    m3 = jnp.min(d3, axis=1, keepdims=True)
    i3 = jnp.min(jnp.where(d3 == m3, itf, kcf), axis=1, keepdims=True)
    return (m1, m2, m3), (i1, i2, i3)


def _dist_body(xn_ref, wnT2_ref, b_ref, a_ref, itf_ref, idx_ref, wb_ref):
    xn = xn_ref[...]
    s2 = jnp.dot(xn, wnT2_ref[...], preferred_element_type=jnp.float32)
    d = (a_ref[...] + b_ref[...]) - s2
    (m1, m2, m3), (i1, i2, i3) = _top3(d, itf_ref[...])
    i1 = i1.astype(jnp.int32)
    i2 = i2.astype(jnp.int32)
    i3 = i3.astype(jnp.int32)
    idx_ref[...] = jnp.concatenate([i1, i2, i3, i3], axis=1)
    inv1 = 1.0 / m1
    inv2 = 1.0 / m2
    inv3 = 1.0 / m3
    tot = (inv1 + inv2) + inv3
    one16 = jnp.ones((xn.shape[0], 16), jnp.float32)
    wb_ref[...] = jnp.concatenate(
        [(inv1 / tot) * one16, (inv2 / tot) * one16, (inv3 / tot) * one16],
        axis=1)


def _sc_combine(x, emb, idx_flat, wb, n_tok):
    info = plsc.get_sparse_core_info()
    mesh = plsc.VectorSubcoreMesh(core_axis_name="c", subcore_axis_name="s")
    tpt = n_tok // NTILES
    nchunk = tpt // CHUNK

    @functools.partial(
        pl.kernel,
        out_type=(jax.ShapeDtypeStruct((n_tok, D), jnp.float32),
                  jax.ShapeDtypeStruct((NTILES, 16), jnp.float32)),
        mesh=mesh,
        scratch_types=[
            pltpu.VMEM((96,), jnp.int32),
            pltpu.VMEM((96,), jnp.int32),
            pltpu.VMEM((3 * CHUNK, D), jnp.float32),
            pltpu.VMEM((CHUNK, D), jnp.float32),
            pltpu.VMEM((CHUNK, D), jnp.float32),
            pltpu.VMEM((CHUNK, 48), jnp.float32),
            pltpu.VMEM((16,), jnp.float32),
            pltpu.SemaphoreType.DMA,
            pltpu.SemaphoreType.DMA,
        ],
    )
    def k(x_hbm, emb_hbm, idxf_hbm, wb_hbm, q_hbm, lp_hbm,
          idxa, idxb, rows, xv, qv, wbv, accv, sem0, sem1):
        wid = lax.axis_index("s") * info.num_cores + lax.axis_index("c")
        accv[...] = jnp.zeros((16,), jnp.float32)

        @pl.loop(0, nchunk)
        def _chunk(c):
            tb = wid * tpt + c * CHUNK
            fb = 3 * tb
            pltpu.sync_copy(idxf_hbm.at[pl.ds(fb, 96)], idxa)
            pltpu.sync_copy(idxf_hbm.at[pl.ds(fb + 96, 96)], idxb)
            cp0 = pltpu.async_copy(emb_hbm.at[idxa], rows.at[pl.ds(0, 96)],
                                   sem0)
            cp1 = pltpu.async_copy(emb_hbm.at[idxb], rows.at[pl.ds(96, 96)],
                                   sem1)
            pltpu.sync_copy(x_hbm.at[pl.ds(tb, CHUNK)], xv)
            pltpu.sync_copy(wb_hbm.at[pl.ds(tb, CHUNK)], wbv)
            cp0.wait()
            cp1.wait()

            @pl.loop(0, CHUNK)
            def _tok(t):
                w0 = wbv[t, pl.ds(0, 16)]
                w1 = wbv[t, pl.ds(16, 16)]
                w2 = wbv[t, pl.ds(32, 16)]
                for v in range(D // 16):
                    sl = pl.ds(v * 16, 16)
                    r0 = rows[3 * t, sl]
                    r1 = rows[3 * t + 1, sl]
                    r2 = rows[3 * t + 2, sl]
                    q = (w0 * r0 + w1 * r1) + w2 * r2
                    xs = xv[t, sl]
                    dq = q - xs
                    qv[t, sl] = xs + dq
                    accv[...] = accv[...] + dq * dq

            pltpu.sync_copy(qv, q_hbm.at[pl.ds(tb, CHUNK)])

        pltpu.sync_copy(accv, lp_hbm.at[wid])

    return k(x, emb, idx_flat, wb)


def kernel(x, emb):
    n_tok = x.shape[0]
    xn = x / jnp.maximum(
        jnp.sqrt(jnp.sum(x * x, axis=1, keepdims=True)), EPS)
    wn = emb / jnp.maximum(
        jnp.sqrt(jnp.sum(emb * emb, axis=1, keepdims=True)), EPS)
    a = jnp.sum(xn ** 2, axis=1, keepdims=True)
    b = jnp.sum(wn ** 2, axis=1).reshape(1, KC)
    wnT2 = wn.T * 2.0
    itf = lax.broadcasted_iota(jnp.float32, (1, KC), 1)
    idx4, wb = pl.pallas_call(
        _dist_body,
        grid=(n_tok // BN,),
        in_specs=[pl.BlockSpec((BN, D), lambda i: (i, 0)),
                  pl.BlockSpec((D, KC), lambda i: (0, 0)),
                  pl.BlockSpec((1, KC), lambda i: (0, 0)),
                  pl.BlockSpec((BN, 1), lambda i: (i, 0)),
                  pl.BlockSpec((1, KC), lambda i: (0, 0))],
        out_specs=[pl.BlockSpec((BN, 4), lambda i: (i, 0)),
                   pl.BlockSpec((BN, 48), lambda i: (i, 0))],
        out_shape=(jax.ShapeDtypeStruct((n_tok, 4), jnp.int32),
                   jax.ShapeDtypeStruct((n_tok, 48), jnp.float32)),
    )(xn, wnT2, b, a, itf)
    top_idx = idx4[:, :3]
    idx_flat = top_idx.reshape(-1)
    q_st, lp = _sc_combine(x, emb, idx_flat, wb, n_tok)
    m = jnp.sum(lp) / jnp.float32(N * D)
    loss = m + CCOST * m
    return (q_st, loss, top_idx)


# speedup vs baseline: 1.3533x; 1.0582x over previous
import functools

import jax
import jax.numpy as jnp
from jax import lax
from jax.experimental import pallas as pl
from jax.experimental.pallas import tpu as pltpu
from jax.experimental.pallas import tpu_sc as plsc

D = 256
KC = 8192
KH = KC // 2
N = 16384
CCOST = 0.25
EPS = 1e-12

BN = 256
NTILES = 32
CHUNK = 64


def _top3(d, itf):
    inf = jnp.float32(jnp.inf)
    kcf = jnp.float32(d.shape[1])
    m1 = jnp.min(d, axis=1, keepdims=True)
    i1 = jnp.min(jnp.where(d == m1, itf, kcf), axis=1, keepdims=True)
    d2 = jnp.where(itf == i1, inf, d)
    m2 = jnp.min(d2, axis=1, keepdims=True)
    i2 = jnp.min(jnp.where(d2 == m2, itf, kcf), axis=1, keepdims=True)
    d3 = jnp.where(itf == i2, inf, d2)
    m3 = jnp.min(d3, axis=1, keepdims=True)
    i3 = jnp.min(jnp.where(d3 == m3, itf, kcf), axis=1, keepdims=True)
    return (m1, m2, m3), (i1, i2, i3)


def _dist_body(xn_ref, wnT2_ref, b_ref, a_ref, itf_ref, idx_ref, wb_ref):
    xn = xn_ref[...]
    s2 = jnp.dot(xn, wnT2_ref[...], preferred_element_type=jnp.float32)
    d = (a_ref[...] + b_ref[...]) - s2
    (m1, m2, m3), (i1, i2, i3) = _top3(d, itf_ref[...])
    i1 = i1.astype(jnp.int32)
    i2 = i2.astype(jnp.int32)
    i3 = i3.astype(jnp.int32)
    idx_ref[...] = jnp.concatenate([i1, i2, i3, i3], axis=1)
    inv1 = 1.0 / m1
    inv2 = 1.0 / m2
    inv3 = 1.0 / m3
    tot = (inv1 + inv2) + inv3
    one16 = jnp.ones((xn.shape[0], 16), jnp.float32)
    wb_ref[...] = jnp.concatenate(
        [(inv1 / tot) * one16, (inv2 / tot) * one16, (inv3 / tot) * one16],
        axis=1)


def _sc_combine(x, emb, idx_flat, wb, n_tok):
    info = plsc.get_sparse_core_info()
    mesh = plsc.VectorSubcoreMesh(core_axis_name="c", subcore_axis_name="s")
    tpt = n_tok // NTILES
    nchunk = tpt // CHUNK

    @functools.partial(
        pl.kernel,
        out_type=(jax.ShapeDtypeStruct((n_tok, D), jnp.float32),
                  jax.ShapeDtypeStruct((NTILES, 16), jnp.float32)),
        mesh=mesh,
        scratch_types=[
            pltpu.VMEM((96,), jnp.int32),
            pltpu.VMEM((96,), jnp.int32),
            pltpu.VMEM((3 * CHUNK, D), jnp.float32),
            pltpu.VMEM((CHUNK, D), jnp.float32),
            pltpu.VMEM((CHUNK, D), jnp.float32),
            pltpu.VMEM((CHUNK, 48), jnp.float32),
            pltpu.VMEM((16,), jnp.float32),
            pltpu.SemaphoreType.DMA,
            pltpu.SemaphoreType.DMA,
        ],
    )
    def k(x_hbm, emb_hbm, idxf_hbm, wb_hbm, q_hbm, lp_hbm,
          idxa, idxb, rows, xv, qv, wbv, accv, sem0, sem1):
        wid = lax.axis_index("s") * info.num_cores + lax.axis_index("c")
        accv[...] = jnp.zeros((16,), jnp.float32)

        @pl.loop(0, nchunk)
        def _chunk(c):
            tb = wid * tpt + c * CHUNK
            fb = 3 * tb
            pltpu.sync_copy(idxf_hbm.at[pl.ds(fb, 96)], idxa)
            pltpu.sync_copy(idxf_hbm.at[pl.ds(fb + 96, 96)], idxb)
            cp0 = pltpu.async_copy(emb_hbm.at[idxa], rows.at[pl.ds(0, 96)],
                                   sem0)
            cp1 = pltpu.async_copy(emb_hbm.at[idxb], rows.at[pl.ds(96, 96)],
                                   sem1)
            pltpu.sync_copy(x_hbm.at[pl.ds(tb, CHUNK)], xv)
            pltpu.sync_copy(wb_hbm.at[pl.ds(tb, CHUNK)], wbv)
            cp0.wait()
            cp1.wait()

            @pl.loop(0, CHUNK)
            def _tok(t):
                w0 = wbv[t, pl.ds(0, 16)]
                w1 = wbv[t, pl.ds(16, 16)]
                w2 = wbv[t, pl.ds(32, 16)]
                for v in range(D // 16):
                    sl = pl.ds(v * 16, 16)
                    r0 = rows[3 * t, sl]
                    r1 = rows[3 * t + 1, sl]
                    r2 = rows[3 * t + 2, sl]
                    q = (w0 * r0 + w1 * r1) + w2 * r2
                    xs = xv[t, sl]
                    dq = q - xs
                    qv[t, sl] = xs + dq
                    accv[...] = accv[...] + dq * dq

            pltpu.sync_copy(qv, q_hbm.at[pl.ds(tb, CHUNK)])

        pltpu.sync_copy(accv, lp_hbm.at[wid])

    return k(x, emb, idx_flat, wb)


def kernel(x, emb):
    n_tok = x.shape[0]
    xn = x / jnp.maximum(
        jnp.sqrt(jnp.sum(x * x, axis=1, keepdims=True)), EPS)
    wn = emb / jnp.maximum(
        jnp.sqrt(jnp.sum(emb * emb, axis=1, keepdims=True)), EPS)
    a = jnp.sum(xn ** 2, axis=1, keepdims=True)
    b = jnp.sum(wn ** 2, axis=1).reshape(1, KC)
    wnT2 = wn.T * 2.0
    itf = lax.broadcasted_iota(jnp.float32, (1, KC), 1)

    nsplit = 2
    ns = n_tok // nsplit
    q_parts, idx_parts, lp_parts = [], [], []
    for kpart in range(nsplit):
        lo = kpart * ns
        xn_k = lax.slice_in_dim(xn, lo, lo + ns, axis=0)
        a_k = lax.slice_in_dim(a, lo, lo + ns, axis=0)
        x_k = lax.slice_in_dim(x, lo, lo + ns, axis=0)
        idx4, wb = pl.pallas_call(
            _dist_body,
            grid=(ns // BN,),
            in_specs=[pl.BlockSpec((BN, D), lambda i: (i, 0)),
                      pl.BlockSpec((D, KC), lambda i: (0, 0)),
                      pl.BlockSpec((1, KC), lambda i: (0, 0)),
                      pl.BlockSpec((BN, 1), lambda i: (i, 0)),
                      pl.BlockSpec((1, KC), lambda i: (0, 0))],
            out_specs=[pl.BlockSpec((BN, 4), lambda i: (i, 0)),
                       pl.BlockSpec((BN, 48), lambda i: (i, 0))],
            out_shape=(jax.ShapeDtypeStruct((ns, 4), jnp.int32),
                       jax.ShapeDtypeStruct((ns, 48), jnp.float32)),
        )(xn_k, wnT2, b, a_k, itf)
        top_idx_k = idx4[:, :3]
        q_k, lp_k = _sc_combine(x_k, emb, top_idx_k.reshape(-1), wb, ns)
        q_parts.append(q_k)
        idx_parts.append(top_idx_k)
        lp_parts.append(lp_k)
    q_st = jnp.concatenate(q_parts, axis=0)
    top_idx = jnp.concatenate(idx_parts, axis=0)
    m = sum(jnp.sum(lp) for lp in lp_parts) / jnp.float32(N * D)
    loss = m + CCOST * m
    return (q_st, loss, top_idx)


# speedup vs baseline: 1.4079x; 1.0403x over previous
import functools

import jax
import jax.numpy as jnp
from jax import lax
from jax.experimental import pallas as pl
from jax.experimental.pallas import tpu as pltpu
from jax.experimental.pallas import tpu_sc as plsc

D = 256
KC = 8192
KH = KC // 2
N = 16384
CCOST = 0.25
EPS = 1e-12

BN = 256
NTILES = 32
CHUNK = 64


def _top3(d, itf):
    inf = jnp.float32(jnp.inf)
    kcf = jnp.float32(d.shape[1])
    m1 = jnp.min(d, axis=1, keepdims=True)
    i1 = jnp.min(jnp.where(d == m1, itf, kcf), axis=1, keepdims=True)
    d2 = jnp.where(itf == i1, inf, d)
    m2 = jnp.min(d2, axis=1, keepdims=True)
    i2 = jnp.min(jnp.where(d2 == m2, itf, kcf), axis=1, keepdims=True)
    d3 = jnp.where(itf == i2, inf, d2)
    m3 = jnp.min(d3, axis=1, keepdims=True)
    i3 = jnp.min(jnp.where(d3 == m3, itf, kcf), axis=1, keepdims=True)
    return (m1, m2, m3), (i1, i2, i3)


def _dist_body(xn_ref, wnT2_ref, b_ref, a_ref, itf_ref, idx_ref, wb_ref):
    xn = xn_ref[...]
    s2 = jnp.dot(xn, wnT2_ref[...], preferred_element_type=jnp.float32)
    d = (a_ref[...] + b_ref[...]) - s2
    (m1, m2, m3), (i1, i2, i3) = _top3(d, itf_ref[...])
    i1 = i1.astype(jnp.int32)
    i2 = i2.astype(jnp.int32)
    i3 = i3.astype(jnp.int32)
    idx_ref[...] = jnp.concatenate([i1, i2, i3, i3], axis=1)
    inv1 = 1.0 / m1
    inv2 = 1.0 / m2
    inv3 = 1.0 / m3
    tot = (inv1 + inv2) + inv3
    one16 = jnp.ones((xn.shape[0], 16), jnp.float32)
    wb_ref[...] = jnp.concatenate(
        [(inv1 / tot) * one16, (inv2 / tot) * one16, (inv3 / tot) * one16],
        axis=1)


def _sc_combine(x, emb, idx_flat, wb, n_tok):
    info = plsc.get_sparse_core_info()
    mesh = plsc.VectorSubcoreMesh(core_axis_name="c", subcore_axis_name="s")
    tpt = n_tok // NTILES
    nchunk = tpt // CHUNK

    @functools.partial(
        pl.kernel,
        out_type=(jax.ShapeDtypeStruct((n_tok, D), jnp.float32),
                  jax.ShapeDtypeStruct((NTILES, 16), jnp.float32)),
        mesh=mesh,
        scratch_types=[
            pltpu.VMEM((96,), jnp.int32),
            pltpu.VMEM((96,), jnp.int32),
            pltpu.VMEM((3 * CHUNK, D), jnp.float32),
            pltpu.VMEM((CHUNK, D), jnp.float32),
            pltpu.VMEM((CHUNK, D), jnp.float32),
            pltpu.VMEM((CHUNK, 48), jnp.float32),
            pltpu.VMEM((16,), jnp.float32),
            pltpu.SemaphoreType.DMA,
            pltpu.SemaphoreType.DMA,
        ],
    )
    def k(x_hbm, emb_hbm, idxf_hbm, wb_hbm, q_hbm, lp_hbm,
          idxa, idxb, rows, xv, qv, wbv, accv, sem0, sem1):
        wid = lax.axis_index("s") * info.num_cores + lax.axis_index("c")
        accv[...] = jnp.zeros((16,), jnp.float32)

        @pl.loop(0, nchunk)
        def _chunk(c):
            tb = wid * tpt + c * CHUNK
            fb = 3 * tb
            pltpu.sync_copy(idxf_hbm.at[pl.ds(fb, 96)], idxa)
            pltpu.sync_copy(idxf_hbm.at[pl.ds(fb + 96, 96)], idxb)
            cp0 = pltpu.async_copy(emb_hbm.at[idxa], rows.at[pl.ds(0, 96)],
                                   sem0)
            cp1 = pltpu.async_copy(emb_hbm.at[idxb], rows.at[pl.ds(96, 96)],
                                   sem1)
            pltpu.sync_copy(x_hbm.at[pl.ds(tb, CHUNK)], xv)
            pltpu.sync_copy(wb_hbm.at[pl.ds(tb, CHUNK)], wbv)
            cp0.wait()
            cp1.wait()

            @pl.loop(0, CHUNK)
            def _tok(t):
                w0 = wbv[t, pl.ds(0, 16)]
                w1 = wbv[t, pl.ds(16, 16)]
                w2 = wbv[t, pl.ds(32, 16)]
                for v in range(D // 16):
                    sl = pl.ds(v * 16, 16)
                    r0 = rows[3 * t, sl]
                    r1 = rows[3 * t + 1, sl]
                    r2 = rows[3 * t + 2, sl]
                    q = (w0 * r0 + w1 * r1) + w2 * r2
                    xs = xv[t, sl]
                    dq = q - xs
                    qv[t, sl] = xs + dq
                    accv[...] = accv[...] + dq * dq

            pltpu.sync_copy(qv, q_hbm.at[pl.ds(tb, CHUNK)])

        pltpu.sync_copy(accv, lp_hbm.at[wid])

    return k(x, emb, idx_flat, wb)


def kernel(x, emb):
    n_tok = x.shape[0]
    xn = x / jnp.maximum(
        jnp.sqrt(jnp.sum(x * x, axis=1, keepdims=True)), EPS)
    wn = emb / jnp.maximum(
        jnp.sqrt(jnp.sum(emb * emb, axis=1, keepdims=True)), EPS)
    a = jnp.sum(xn ** 2, axis=1, keepdims=True)
    b = jnp.sum(wn ** 2, axis=1).reshape(1, KC)
    wnT2 = wn.T * 2.0
    itf = lax.broadcasted_iota(jnp.float32, (1, KC), 1)

    nsplit = 4
    ns = n_tok // nsplit
    q_parts, idx_parts, lp_parts = [], [], []
    for kpart in range(nsplit):
        lo = kpart * ns
        xn_k = lax.slice_in_dim(xn, lo, lo + ns, axis=0)
        a_k = lax.slice_in_dim(a, lo, lo + ns, axis=0)
        x_k = lax.slice_in_dim(x, lo, lo + ns, axis=0)
        idx4, wb = pl.pallas_call(
            _dist_body,
            grid=(ns // BN,),
            in_specs=[pl.BlockSpec((BN, D), lambda i: (i, 0)),
                      pl.BlockSpec((D, KC), lambda i: (0, 0)),
                      pl.BlockSpec((1, KC), lambda i: (0, 0)),
                      pl.BlockSpec((BN, 1), lambda i: (i, 0)),
                      pl.BlockSpec((1, KC), lambda i: (0, 0))],
            out_specs=[pl.BlockSpec((BN, 4), lambda i: (i, 0)),
                       pl.BlockSpec((BN, 48), lambda i: (i, 0))],
            out_shape=(jax.ShapeDtypeStruct((ns, 4), jnp.int32),
                       jax.ShapeDtypeStruct((ns, 48), jnp.float32)),
        )(xn_k, wnT2, b, a_k, itf)
        top_idx_k = idx4[:, :3]
        q_k, lp_k = _sc_combine(x_k, emb, top_idx_k.reshape(-1), wb, ns)
        q_parts.append(q_k)
        idx_parts.append(top_idx_k)
        lp_parts.append(lp_k)
    q_st = jnp.concatenate(q_parts, axis=0)
    top_idx = jnp.concatenate(idx_parts, axis=0)
    m = sum(jnp.sum(lp) for lp in lp_parts) / jnp.float32(N * D)
    loss = m + CCOST * m
    return (q_st, loss, top_idx)
